# baseline (speedup 1.0000x reference)
import functools

import numpy as np
import jax
import jax.numpy as jnp
from jax import lax
from jax.experimental import pallas as pl
from jax.experimental.pallas import tpu as pltpu
from jax.experimental.pallas import tpu_sc as plsc

_B, _L, _C = 4096, 200, 16
_V, _P, _M = 100000, 45, 2000
_N = _B * _L
_NC, _NS = 2, 16
_NW = _NC * _NS
_CHUNK = _N // _NW
_BROWS = _B // _NW
_NB = 4
_K = _NB * _L
_NBLK = _BROWS // _NB
_GSPLIT = (0, 128, 200)
_NGRP = _K // 16

_SPAN = np.uint32(_M)
_MULT = np.uint32((pow(2, 16, _M) ** 2) % _M)
_TAB = 64


def _np_threefry_block(k0, k1, x0, x1):
    k0 = np.uint32(k0)
    k1 = np.uint32(k1)
    ks = [k0, k1, np.uint32(k0 ^ k1 ^ np.uint32(0x1BD11BDA))]
    rots = [(13, 15, 26, 6), (17, 29, 16, 24)]
    x = [np.uint32(x0 + ks[0]), np.uint32(x1 + ks[1])]
    for g in range(5):
        for r in rots[g % 2]:
            x[0] = np.uint32(x[0] + x[1])
            x[1] = np.uint32((np.uint32(x[1] << np.uint32(r))) |
                             (x[1] >> np.uint32(32 - r)))
            x[1] = np.uint32(x[0] ^ x[1])
        x[0] = np.uint32(x[0] + ks[(g + 1) % 3])
        x[1] = np.uint32(x[1] + ks[(g + 2) % 3] + np.uint32(g + 1))
    return x[0], x[1]


def _make_key_tables():
    a0 = np.zeros(_TAB, np.uint32)
    a1 = np.zeros(_TAB, np.uint32)
    b0 = np.zeros(_TAB, np.uint32)
    b1 = np.zeros(_TAB, np.uint32)
    for p in range(_P):
        f0, f1 = _np_threefry_block(0, 42, 0, p)
        a0[p], a1[p] = _np_threefry_block(f0, f1, 0, 0)
        b0[p], b1[p] = _np_threefry_block(f0, f1, 0, 1)
    return (a0.view(np.int32), a1.view(np.int32),
            b0.view(np.int32), b1.view(np.int32))


_A0, _A1, _B0, _B1 = _make_key_tables()


def _tf_xor(k0, k1, cnt):
    ks = (k0, k1, k0 ^ k1 ^ jnp.uint32(0x1BD11BDA))
    x0 = k0
    x1 = cnt + k1
    rots = ((13, 15, 26, 6), (17, 29, 16, 24))
    for g in range(5):
        for r in rots[g % 2]:
            x0 = x0 + x1
            x1 = lax.shift_left(x1, jnp.uint32(r)) | lax.shift_right_logical(
                x1, jnp.uint32(32 - r))
            x1 = x0 ^ x1
        x0 = x0 + ks[(g + 1) % 3]
        x1 = x1 + ks[(g + 2) % 3] + jnp.uint32(g + 1)
    return x0 ^ x1


def _sc_body(pos_hbm, word_hbm, a0_hbm, a1_hbm, b0_hbm, b1_hbm, pri_hbm,
             tgt_hbm, lut_hbm,
             obfw_hbm, char_hbm, prim_hbm, eq_hbm,
             a0_v, a1_v, b0_v, b1_v, pri_v, tgt_v,
             pos_v, word_v, obfw_v, prim_v, eq_v, char_v, gsem):
    wid = lax.axis_index("s") * _NC + lax.axis_index("c")

    pltpu.sync_copy(a0_hbm, a0_v)
    pltpu.sync_copy(a1_hbm, a1_v)
    pltpu.sync_copy(b0_hbm, b0_v)
    pltpu.sync_copy(b1_hbm, b1_v)
    pltpu.sync_copy(pri_hbm, pri_v)
    pltpu.sync_copy(tgt_hbm, tgt_v)

    def block_body(blk, carry):
        base = wid * _CHUNK + blk * _K
        b0 = wid * _BROWS + blk * _NB
        pltpu.sync_copy(pos_hbm.at[pl.ds(base, _K)], pos_v)
        pltpu.sync_copy(word_hbm.at[pl.ds(base, _K)], word_v)

        @plsc.parallel_loop(0, _NGRP, step=1, unroll=4)
        def grp_body(i):
            o = i * 16
            p = pos_v[pl.ds(o, 16)]
            k_a0 = plsc.bitcast(plsc.load_gather(a0_v, [p]), jnp.uint32)
            k_a1 = plsc.bitcast(plsc.load_gather(a1_v, [p]), jnp.uint32)
            k_b0 = plsc.bitcast(plsc.load_gather(b0_v, [p]), jnp.uint32)
            k_b1 = plsc.bitcast(plsc.load_gather(b1_v, [p]), jnp.uint32)
            prg = plsc.load_gather(pri_v, [p])
            cnt = plsc.bitcast(
                base + o + lax.broadcasted_iota(jnp.int32, (16,), 0),
                jnp.uint32)
            hi = _tf_xor(k_a0, k_a1, cnt)
            lo = _tf_xor(k_b0, k_b1, cnt)
            off = lax.rem(lax.rem(hi, _SPAN) * _MULT + lax.rem(lo, _SPAN),
                          _SPAN)
            idxf = p * jnp.int32(_M) + plsc.bitcast(off, jnp.int32)
            w = plsc.load_gather(tgt_v, [idxf])
            wd = word_v[pl.ds(o, 16)]
            obfw_v[pl.ds(o, 16)] = w
            prim_v[pl.ds(o, 16)] = prg
            eq_v[pl.ds(o, 16)] = jnp.where(w == wd, jnp.int32(1), jnp.int32(0))

        descs = [
            pltpu.async_copy(
                lut_hbm.at[obfw_v.at[pl.ds(bi * _L + l0, l1 - l0)]],
                char_v.at[bi, pl.ds(l0, l1 - l0)], gsem)
            for bi in range(_NB)
            for l0, l1 in zip(_GSPLIT[:-1], _GSPLIT[1:])
        ]
        for d in descs:
            d.wait()

        pltpu.sync_copy(obfw_v, obfw_hbm.at[pl.ds(base, _K)])
        pltpu.sync_copy(char_v, char_hbm.at[pl.ds(b0, _NB)])
        pltpu.sync_copy(prim_v, prim_hbm.at[pl.ds(base, _K)])
        pltpu.sync_copy(eq_v, eq_hbm.at[pl.ds(base, _K)])
        return carry

    lax.fori_loop(0, _NBLK, block_body, 0)


@functools.cache
def _sc_call():
  return pl.kernel(
    _sc_body,
    mesh=plsc.VectorSubcoreMesh(core_axis_name="c", subcore_axis_name="s",
                                num_cores=_NC, num_subcores=_NS),
    out_type=[
        jax.ShapeDtypeStruct((_N,), jnp.int32),
        jax.ShapeDtypeStruct((_B, _L, _C), jnp.int32),
        jax.ShapeDtypeStruct((_N,), jnp.int32),
        jax.ShapeDtypeStruct((_N,), jnp.int32),
    ],
    scratch_types=[
        pltpu.VMEM((_TAB,), jnp.int32),
        pltpu.VMEM((_TAB,), jnp.int32),
        pltpu.VMEM((_TAB,), jnp.int32),
        pltpu.VMEM((_TAB,), jnp.int32),
        pltpu.VMEM((_TAB,), jnp.int32),
        pltpu.VMEM((_P * _M,), jnp.int32),
        pltpu.VMEM((_K,), jnp.int32),
        pltpu.VMEM((_K,), jnp.int32),
        pltpu.VMEM((_K,), jnp.int32),
        pltpu.VMEM((_K,), jnp.int32),
        pltpu.VMEM((_K,), jnp.int32),
        pltpu.VMEM((_NB, _L, _C), jnp.int32),
        pltpu.SemaphoreType.DMA,
    ],
    compiler_params=pltpu.CompilerParams(needs_layout_passes=False,
                                         use_tc_tiling_on_sc=False),
  )


def _sc_run(pos_f, word_f, a0, a1, b0, b1, pri_tab, tgt_f, lut):
    return _sc_call()(pos_f, word_f, a0, a1, b0, b1, pri_tab, tgt_f, lut)


def kernel(inp_word, inp_char, inp_pos, inp_mask, tgtwords, pri_term_mask, lut):
    pos_f = inp_pos.reshape(_N)
    word_f = inp_word.reshape(_N)
    pri_tab = jnp.zeros((_TAB,), jnp.int32).at[:_P].set(
        pri_term_mask.astype(jnp.int32))
    tgt_f = tgtwords.reshape(_P * _M)

    obfw_f, char_f, prim_f, eq_f = _sc_run(
        pos_f, word_f, jnp.asarray(_A0), jnp.asarray(_A1), jnp.asarray(_B0),
        jnp.asarray(_B1), pri_tab, tgt_f, lut)

    obf_word = obfw_f.reshape(_B, _L)
    obf_char = char_f
    pri_mask = prim_f.reshape(_B, _L).astype(bool)
    cpy_mask = inp_mask & eq_f.reshape(_B, _L).astype(bool)
    obf_mask = jnp.ones((_B, _L), bool)
    return (obf_word, inp_word, obf_char, inp_pos, obf_mask, pri_mask,
            cpy_mask)


# speedup vs baseline: 1.4419x; 1.4419x over previous
import functools

import numpy as np
import jax
import jax.numpy as jnp
from jax import lax
from jax.experimental import pallas as pl
from jax.experimental.pallas import tpu as pltpu
from jax.experimental.pallas import tpu_sc as plsc

_B, _L, _C = 4096, 200, 16
_V, _P, _M = 100000, 45, 2000
_N = _B * _L
_NC, _NS = 2, 16
_NW = _NC * _NS
_CHUNK = _N // _NW
_BROWS = _B // _NW
_NB = 4
_K = _NB * _L
_NBLK = _BROWS // _NB
_GSPLIT = (0, 128, 200)
_NGRP = _K // 16

_SPAN = np.uint32(_M)
_MULT = np.uint32((pow(2, 16, _M) ** 2) % _M)
_TAB = 64


def _np_threefry_block(k0, k1, x0, x1):
    k0 = np.uint32(k0)
    k1 = np.uint32(k1)
    ks = [k0, k1, np.uint32(k0 ^ k1 ^ np.uint32(0x1BD11BDA))]
    rots = [(13, 15, 26, 6), (17, 29, 16, 24)]
    x = [np.uint32(x0 + ks[0]), np.uint32(x1 + ks[1])]
    for g in range(5):
        for r in rots[g % 2]:
            x[0] = np.uint32(x[0] + x[1])
            x[1] = np.uint32((np.uint32(x[1] << np.uint32(r))) |
                             (x[1] >> np.uint32(32 - r)))
            x[1] = np.uint32(x[0] ^ x[1])
        x[0] = np.uint32(x[0] + ks[(g + 1) % 3])
        x[1] = np.uint32(x[1] + ks[(g + 2) % 3] + np.uint32(g + 1))
    return x[0], x[1]


def _make_key_tables():
    a0 = np.zeros(_TAB, np.uint32)
    a1 = np.zeros(_TAB, np.uint32)
    b0 = np.zeros(_TAB, np.uint32)
    b1 = np.zeros(_TAB, np.uint32)
    for p in range(_P):
        f0, f1 = _np_threefry_block(0, 42, 0, p)
        a0[p], a1[p] = _np_threefry_block(f0, f1, 0, 0)
        b0[p], b1[p] = _np_threefry_block(f0, f1, 0, 1)
    return (a0.view(np.int32), a1.view(np.int32),
            b0.view(np.int32), b1.view(np.int32))


_A0, _A1, _B0, _B1 = _make_key_tables()


def _tf_xor(k0, k1, cnt):
    ks = (k0, k1, k0 ^ k1 ^ jnp.uint32(0x1BD11BDA))
    x0 = k0
    x1 = cnt + k1
    rots = ((13, 15, 26, 6), (17, 29, 16, 24))
    for g in range(5):
        for r in rots[g % 2]:
            x0 = x0 + x1
            x1 = lax.shift_left(x1, jnp.uint32(r)) | lax.shift_right_logical(
                x1, jnp.uint32(32 - r))
            x1 = x0 ^ x1
        x0 = x0 + ks[(g + 1) % 3]
        x1 = x1 + ks[(g + 2) % 3] + jnp.uint32(g + 1)
    return x0 ^ x1


def _sc_body(pos_hbm, word_hbm, a0_hbm, a1_hbm, b0_hbm, b1_hbm, pri_hbm,
             tgt_hbm, lut_hbm,
             obfw_hbm, char_hbm, prim_hbm, eq_hbm,
             a0_v, a1_v, b0_v, b1_v, pri_v, tgt_v,
             pos_v, word_v, obfw_v, prim_v, eq_v, char_v, gsem):
    wid = lax.axis_index("s") * _NC + lax.axis_index("c")

    pltpu.sync_copy(a0_hbm, a0_v)
    pltpu.sync_copy(a1_hbm, a1_v)
    pltpu.sync_copy(b0_hbm, b0_v)
    pltpu.sync_copy(b1_hbm, b1_v)
    pltpu.sync_copy(pri_hbm, pri_v)
    pltpu.sync_copy(tgt_hbm, tgt_v)

    def block_body(blk, carry):
        base = wid * _CHUNK + blk * _K
        b0 = wid * _BROWS + blk * _NB
        pltpu.sync_copy(pos_hbm.at[pl.ds(base, _K)], pos_v)
        pltpu.sync_copy(word_hbm.at[pl.ds(base, _K)], word_v)

        @plsc.parallel_loop(0, _NGRP, step=1, unroll=4)
        def grp_body(i):
            o = i * 16
            p = pos_v[pl.ds(o, 16)]
            k_a0 = plsc.bitcast(plsc.load_gather(a0_v, [p]), jnp.uint32)
            k_a1 = plsc.bitcast(plsc.load_gather(a1_v, [p]), jnp.uint32)
            k_b0 = plsc.bitcast(plsc.load_gather(b0_v, [p]), jnp.uint32)
            k_b1 = plsc.bitcast(plsc.load_gather(b1_v, [p]), jnp.uint32)
            prg = plsc.load_gather(pri_v, [p])
            cnt = plsc.bitcast(
                base + o + lax.broadcasted_iota(jnp.int32, (16,), 0),
                jnp.uint32)
            hi = _tf_xor(k_a0, k_a1, cnt)
            lo = _tf_xor(k_b0, k_b1, cnt)
            off = lax.rem(lax.rem(hi, _SPAN) * _MULT + lax.rem(lo, _SPAN),
                          _SPAN)
            idxf = p * jnp.int32(_M) + plsc.bitcast(off, jnp.int32)
            w = plsc.load_gather(tgt_v, [idxf])
            wd = word_v[pl.ds(o, 16)]
            obfw_v[pl.ds(o, 16)] = w
            prim_v[pl.ds(o, 16)] = prg
            eq_v[pl.ds(o, 16)] = jnp.where(w == wd, jnp.int32(1), jnp.int32(0))

        descs = [
            pltpu.async_copy(
                lut_hbm.at[obfw_v.at[pl.ds(bi * _L + l0, l1 - l0)]],
                char_v.at[bi, pl.ds(l0, l1 - l0)], gsem)
            for bi in range(_NB)
            for l0, l1 in zip(_GSPLIT[:-1], _GSPLIT[1:])
        ]
        for d in descs:
            d.wait()

        pltpu.sync_copy(obfw_v, obfw_hbm.at[pl.ds(base, _K)])
        pltpu.sync_copy(char_v, char_hbm.at[pl.ds(b0, _NB), :, pl.ds(0, _C)])
        pltpu.sync_copy(prim_v, prim_hbm.at[pl.ds(base, _K)])
        pltpu.sync_copy(eq_v, eq_hbm.at[pl.ds(base, _K)])
        return carry

    lax.fori_loop(0, _NBLK, block_body, 0)


@functools.cache
def _sc_call():
  return pl.kernel(
    _sc_body,
    mesh=plsc.VectorSubcoreMesh(core_axis_name="c", subcore_axis_name="s",
                                num_cores=_NC, num_subcores=_NS),
    out_type=[
        jax.ShapeDtypeStruct((_N,), jnp.int32),
        jax.ShapeDtypeStruct((_B, _L, 128), jnp.int32),
        jax.ShapeDtypeStruct((_N,), jnp.int32),
        jax.ShapeDtypeStruct((_N,), jnp.int32),
    ],
    scratch_types=[
        pltpu.VMEM((_TAB,), jnp.int32),
        pltpu.VMEM((_TAB,), jnp.int32),
        pltpu.VMEM((_TAB,), jnp.int32),
        pltpu.VMEM((_TAB,), jnp.int32),
        pltpu.VMEM((_TAB,), jnp.int32),
        pltpu.VMEM((_P * _M,), jnp.int32),
        pltpu.VMEM((_K,), jnp.int32),
        pltpu.VMEM((_K,), jnp.int32),
        pltpu.VMEM((_K,), jnp.int32),
        pltpu.VMEM((_K,), jnp.int32),
        pltpu.VMEM((_K,), jnp.int32),
        pltpu.VMEM((_NB, _L, _C), jnp.int32),
        pltpu.SemaphoreType.DMA,
    ],
    compiler_params=pltpu.CompilerParams(needs_layout_passes=False,
                                         use_tc_tiling_on_sc=False),
  )


def _sc_run(pos_f, word_f, a0, a1, b0, b1, pri_tab, tgt_f, lut):
    return _sc_call()(pos_f, word_f, a0, a1, b0, b1, pri_tab, tgt_f, lut)


def kernel(inp_word, inp_char, inp_pos, inp_mask, tgtwords, pri_term_mask, lut):
    pos_f = inp_pos.reshape(_N)
    word_f = inp_word.reshape(_N)
    pri_tab = jnp.zeros((_TAB,), jnp.int32).at[:_P].set(
        pri_term_mask.astype(jnp.int32))
    tgt_f = tgtwords.reshape(_P * _M)

    obfw_f, char_f, prim_f, eq_f = _sc_run(
        pos_f, word_f, jnp.asarray(_A0), jnp.asarray(_A1), jnp.asarray(_B0),
        jnp.asarray(_B1), pri_tab, tgt_f, lut)

    obf_word = obfw_f.reshape(_B, _L)
    obf_char = char_f[:, :, :_C]
    pri_mask = prim_f.reshape(_B, _L).astype(bool)
    cpy_mask = inp_mask & eq_f.reshape(_B, _L).astype(bool)
    obf_mask = jnp.ones((_B, _L), bool)
    return (obf_word, inp_word, obf_char, inp_pos, obf_mask, pri_mask,
            cpy_mask)


# speedup vs baseline: 1.6705x; 1.1585x over previous
import functools

import numpy as np
import jax
import jax.numpy as jnp
from jax import lax
from jax.experimental import pallas as pl
from jax.experimental.pallas import tpu as pltpu
from jax.experimental.pallas import tpu_sc as plsc

_B, _L, _C = 4096, 200, 16
_V, _P, _M = 100000, 45, 2000
_N = _B * _L
_NC, _NS = 2, 16
_NW = _NC * _NS
_CHUNK = _N // _NW
_BROWS = _B // _NW
_NB = 4
_K = _NB * _L
_NBLK = _BROWS // _NB
_GSPLIT = (0, 128, 200)
_NGRP = _K // 16

_SPAN = np.uint32(_M)
_MULT = np.uint32((pow(2, 16, _M) ** 2) % _M)
_TAB = 64


def _np_threefry_block(k0, k1, x0, x1):
    k0 = np.uint32(k0)
    k1 = np.uint32(k1)
    ks = [k0, k1, np.uint32(k0 ^ k1 ^ np.uint32(0x1BD11BDA))]
    rots = [(13, 15, 26, 6), (17, 29, 16, 24)]
    x = [np.uint32(x0 + ks[0]), np.uint32(x1 + ks[1])]
    for g in range(5):
        for r in rots[g % 2]:
            x[0] = np.uint32(x[0] + x[1])
            x[1] = np.uint32((np.uint32(x[1] << np.uint32(r))) |
                             (x[1] >> np.uint32(32 - r)))
            x[1] = np.uint32(x[0] ^ x[1])
        x[0] = np.uint32(x[0] + ks[(g + 1) % 3])
        x[1] = np.uint32(x[1] + ks[(g + 2) % 3] + np.uint32(g + 1))
    return x[0], x[1]


def _make_key_tables():
    a0 = np.zeros(_TAB, np.uint32)
    a1 = np.zeros(_TAB, np.uint32)
    b0 = np.zeros(_TAB, np.uint32)
    b1 = np.zeros(_TAB, np.uint32)
    for p in range(_P):
        f0, f1 = _np_threefry_block(0, 42, 0, p)
        a0[p], a1[p] = _np_threefry_block(f0, f1, 0, 0)
        b0[p], b1[p] = _np_threefry_block(f0, f1, 0, 1)
    return (a0.view(np.int32), a1.view(np.int32),
            b0.view(np.int32), b1.view(np.int32))


_A0, _A1, _B0, _B1 = _make_key_tables()


def _tf_xor(k0, k1, cnt):
    ks = (k0, k1, k0 ^ k1 ^ jnp.uint32(0x1BD11BDA))
    x0 = k0
    x1 = cnt + k1
    rots = ((13, 15, 26, 6), (17, 29, 16, 24))
    for g in range(5):
        for r in rots[g % 2]:
            x0 = x0 + x1
            x1 = lax.shift_left(x1, jnp.uint32(r)) | lax.shift_right_logical(
                x1, jnp.uint32(32 - r))
            x1 = x0 ^ x1
        x0 = x0 + ks[(g + 1) % 3]
        x1 = x1 + ks[(g + 2) % 3] + jnp.uint32(g + 1)
    return x0 ^ x1


def _sc_body(pos_hbm, word_hbm, a0_hbm, a1_hbm, b0_hbm, b1_hbm, pri_hbm,
             tgt_hbm, lut_hbm,
             obfw_hbm, char_hbm, prim_hbm, eq_hbm,
             a0_v, a1_v, b0_v, b1_v, pri_v, tgt_v,
             pos_v, word_v, obfw_v, prim_v, eq_v, char_v,
             isem, gsem, osem0, osem1):
    wid = lax.axis_index("s") * _NC + lax.axis_index("c")
    osems = (osem0, osem1)

    pltpu.sync_copy(a0_hbm, a0_v)
    pltpu.sync_copy(a1_hbm, a1_v)
    pltpu.sync_copy(b0_hbm, b0_v)
    pltpu.sync_copy(b1_hbm, b1_v)
    pltpu.sync_copy(pri_hbm, pri_v)
    pltpu.sync_copy(tgt_hbm, tgt_v)

    def in_copies(blk, ph):
        base = wid * _CHUNK + blk * _K
        return (
            pltpu.make_async_copy(pos_hbm.at[pl.ds(base, _K)],
                                  pos_v.at[ph], isem),
            pltpu.make_async_copy(word_hbm.at[pl.ds(base, _K)],
                                  word_v.at[ph], isem),
        )

    def out_copies(blk, ph):
        base = wid * _CHUNK + blk * _K
        b0 = wid * _BROWS + blk * _NB
        return (
            pltpu.make_async_copy(obfw_v.at[ph],
                                  obfw_hbm.at[pl.ds(base, _K)], osems[ph]),
            pltpu.make_async_copy(
                char_v.at[ph],
                char_hbm.at[pl.ds(b0, _NB), :, pl.ds(0, _C)], osems[ph]),
            pltpu.make_async_copy(prim_v.at[ph],
                                  prim_hbm.at[pl.ds(base, _K)], osems[ph]),
            pltpu.make_async_copy(eq_v.at[ph],
                                  eq_hbm.at[pl.ds(base, _K)], osems[ph]),
        )

    for d in in_copies(0, 0):
        d.start()

    def do_block(blk, ph):
        @pl.when(blk >= 2)
        def _():
            for d in out_copies(blk - 2, ph):
                d.wait()
        for d in in_copies(blk, ph):
            d.wait()
        for d in in_copies(jnp.minimum(blk + 1, _NBLK - 1), 1 - ph):
            d.start()

        base = wid * _CHUNK + blk * _K

        @plsc.parallel_loop(0, _NGRP, step=1, unroll=4)
        def grp_body(i):
            o = i * 16
            p = pos_v[ph, pl.ds(o, 16)]
            k_a0 = plsc.bitcast(plsc.load_gather(a0_v, [p]), jnp.uint32)
            k_a1 = plsc.bitcast(plsc.load_gather(a1_v, [p]), jnp.uint32)
            k_b0 = plsc.bitcast(plsc.load_gather(b0_v, [p]), jnp.uint32)
            k_b1 = plsc.bitcast(plsc.load_gather(b1_v, [p]), jnp.uint32)
            prg = plsc.load_gather(pri_v, [p])
            cnt = plsc.bitcast(
                base + o + lax.broadcasted_iota(jnp.int32, (16,), 0),
                jnp.uint32)
            hi = _tf_xor(k_a0, k_a1, cnt)
            lo = _tf_xor(k_b0, k_b1, cnt)
            off = lax.rem(lax.rem(hi, _SPAN) * _MULT + lax.rem(lo, _SPAN),
                          _SPAN)
            idxf = p * jnp.int32(_M) + plsc.bitcast(off, jnp.int32)
            w = plsc.load_gather(tgt_v, [idxf])
            wd = word_v[ph, pl.ds(o, 16)]
            obfw_v[ph, pl.ds(o, 16)] = w
            prim_v[ph, pl.ds(o, 16)] = prg
            eq_v[ph, pl.ds(o, 16)] = jnp.where(w == wd, jnp.int32(1),
                                               jnp.int32(0))

        descs = [
            pltpu.async_copy(
                lut_hbm.at[obfw_v.at[ph, pl.ds(bi * _L + l0, l1 - l0)]],
                char_v.at[ph, bi, pl.ds(l0, l1 - l0)], gsem)
            for bi in range(_NB)
            for l0, l1 in zip(_GSPLIT[:-1], _GSPLIT[1:])
        ]
        for d in descs:
            d.wait()
        for d in out_copies(blk, ph):
            d.start()

    def body2(g, carry):
        do_block(g * 2, 0)
        do_block(g * 2 + 1, 1)
        return carry

    lax.fori_loop(0, _NBLK // 2, body2, 0)
    for ph in (0, 1):
        for d in out_copies(_NBLK - 2 + ph, ph):
            d.wait()
    for d in in_copies(_NBLK - 1, 0):
        d.wait()


@functools.cache
def _sc_call():
  return pl.kernel(
    _sc_body,
    mesh=plsc.VectorSubcoreMesh(core_axis_name="c", subcore_axis_name="s",
                                num_cores=_NC, num_subcores=_NS),
    out_type=[
        jax.ShapeDtypeStruct((_N,), jnp.int32),
        jax.ShapeDtypeStruct((_B, _L, 128), jnp.int32),
        jax.ShapeDtypeStruct((_N,), jnp.int32),
        jax.ShapeDtypeStruct((_N,), jnp.int32),
    ],
    scratch_types=[
        pltpu.VMEM((_TAB,), jnp.int32),
        pltpu.VMEM((_TAB,), jnp.int32),
        pltpu.VMEM((_TAB,), jnp.int32),
        pltpu.VMEM((_TAB,), jnp.int32),
        pltpu.VMEM((_TAB,), jnp.int32),
        pltpu.VMEM((_P * _M,), jnp.int32),
        pltpu.VMEM((2, _K), jnp.int32),
        pltpu.VMEM((2, _K), jnp.int32),
        pltpu.VMEM((2, _K), jnp.int32),
        pltpu.VMEM((2, _K), jnp.int32),
        pltpu.VMEM((2, _K), jnp.int32),
        pltpu.VMEM((2, _NB, _L, _C), jnp.int32),
        pltpu.SemaphoreType.DMA,
        pltpu.SemaphoreType.DMA,
        pltpu.SemaphoreType.DMA,
        pltpu.SemaphoreType.DMA,
    ],
    compiler_params=pltpu.CompilerParams(needs_layout_passes=False,
                                         use_tc_tiling_on_sc=False),
  )


def _sc_run(pos_f, word_f, a0, a1, b0, b1, pri_tab, tgt_f, lut):
    return _sc_call()(pos_f, word_f, a0, a1, b0, b1, pri_tab, tgt_f, lut)


def kernel(inp_word, inp_char, inp_pos, inp_mask, tgtwords, pri_term_mask, lut):
    pos_f = inp_pos.reshape(_N)
    word_f = inp_word.reshape(_N)
    pri_tab = jnp.zeros((_TAB,), jnp.int32).at[:_P].set(
        pri_term_mask.astype(jnp.int32))
    tgt_f = tgtwords.reshape(_P * _M)

    obfw_f, char_f, prim_f, eq_f = _sc_run(
        pos_f, word_f, jnp.asarray(_A0), jnp.asarray(_A1), jnp.asarray(_B0),
        jnp.asarray(_B1), pri_tab, tgt_f, lut)

    obf_word = obfw_f.reshape(_B, _L)
    obf_char = char_f[:, :, :_C]
    pri_mask = prim_f.reshape(_B, _L).astype(bool)
    cpy_mask = inp_mask & eq_f.reshape(_B, _L).astype(bool)
    obf_mask = jnp.ones((_B, _L), bool)
    return (obf_word, inp_word, obf_char, inp_pos, obf_mask, pri_mask,
            cpy_mask)


# speedup vs baseline: 1.8181x; 1.0884x over previous
import functools

import numpy as np
import jax
import jax.numpy as jnp
from jax import lax
from jax.experimental import pallas as pl
from jax.experimental.pallas import tpu as pltpu
from jax.experimental.pallas import tpu_sc as plsc

_B, _L, _C = 4096, 200, 16
_V, _P, _M = 100000, 45, 2000
_N = _B * _L
_NC, _NS = 2, 16
_NW = _NC * _NS
_CHUNK = _N // _NW
_BROWS = _B // _NW
_NB = 4
_K = _NB * _L
_NBLK = _BROWS // _NB
_GSPLIT = (0, 128, 200)
_NGRP = _K // 16

_SPAN = np.uint32(_M)
_MULT = np.uint32((pow(2, 16, _M) ** 2) % _M)
_TAB = 64


def _np_threefry_block(k0, k1, x0, x1):
    k0 = np.uint32(k0)
    k1 = np.uint32(k1)
    ks = [k0, k1, np.uint32(k0 ^ k1 ^ np.uint32(0x1BD11BDA))]
    rots = [(13, 15, 26, 6), (17, 29, 16, 24)]
    x = [np.uint32(x0 + ks[0]), np.uint32(x1 + ks[1])]
    for g in range(5):
        for r in rots[g % 2]:
            x[0] = np.uint32(x[0] + x[1])
            x[1] = np.uint32((np.uint32(x[1] << np.uint32(r))) |
                             (x[1] >> np.uint32(32 - r)))
            x[1] = np.uint32(x[0] ^ x[1])
        x[0] = np.uint32(x[0] + ks[(g + 1) % 3])
        x[1] = np.uint32(x[1] + ks[(g + 2) % 3] + np.uint32(g + 1))
    return x[0], x[1]


def _make_key_tables():
    a0 = np.zeros(_TAB, np.uint32)
    a1 = np.zeros(_TAB, np.uint32)
    b0 = np.zeros(_TAB, np.uint32)
    b1 = np.zeros(_TAB, np.uint32)
    for p in range(_P):
        f0, f1 = _np_threefry_block(0, 42, 0, p)
        a0[p], a1[p] = _np_threefry_block(f0, f1, 0, 0)
        b0[p], b1[p] = _np_threefry_block(f0, f1, 0, 1)
    return (a0.view(np.int32), a1.view(np.int32),
            b0.view(np.int32), b1.view(np.int32))


_A0, _A1, _B0, _B1 = _make_key_tables()


def _tf_xor(k0, k1, cnt):
    ks = (k0, k1, k0 ^ k1 ^ jnp.uint32(0x1BD11BDA))
    x0 = k0
    x1 = cnt + k1
    rots = ((13, 15, 26, 6), (17, 29, 16, 24))
    for g in range(5):
        for r in rots[g % 2]:
            x0 = x0 + x1
            x1 = lax.shift_left(x1, jnp.uint32(r)) | lax.shift_right_logical(
                x1, jnp.uint32(32 - r))
            x1 = x0 ^ x1
        x0 = x0 + ks[(g + 1) % 3]
        x1 = x1 + ks[(g + 2) % 3] + jnp.uint32(g + 1)
    return x0 ^ x1


def _sc_body(pos_hbm, a0_hbm, a1_hbm, b0_hbm, b1_hbm, pri_hbm,
             tgt_hbm, lut_hbm,
             obfw_hbm, char_hbm, prim_hbm,
             a0_v, a1_v, b0_v, b1_v, pri_v, tgt_v,
             pos_v, obfw_v, prim_v, char_v,
             isem, gsem, osem0, osem1):
    wid = lax.axis_index("s") * _NC + lax.axis_index("c")
    osems = (osem0, osem1)

    pltpu.sync_copy(a0_hbm, a0_v)
    pltpu.sync_copy(a1_hbm, a1_v)
    pltpu.sync_copy(b0_hbm, b0_v)
    pltpu.sync_copy(b1_hbm, b1_v)
    pltpu.sync_copy(pri_hbm, pri_v)
    pltpu.sync_copy(tgt_hbm, tgt_v)

    def in_copies(blk, ph):
        base = wid * _CHUNK + blk * _K
        return (
            pltpu.make_async_copy(pos_hbm.at[pl.ds(base, _K)],
                                  pos_v.at[ph], isem),
        )

    def out_copies(blk, ph):
        base = wid * _CHUNK + blk * _K
        b0 = wid * _BROWS + blk * _NB
        return (
            pltpu.make_async_copy(obfw_v.at[ph],
                                  obfw_hbm.at[pl.ds(base, _K)], osems[ph]),
            pltpu.make_async_copy(
                char_v.at[ph],
                char_hbm.at[pl.ds(b0, _NB), :, pl.ds(0, _C)], osems[ph]),
            pltpu.make_async_copy(prim_v.at[ph],
                                  prim_hbm.at[pl.ds(base, _K)], osems[ph]),
        )

    def gather_copies(ph):
        return [
            pltpu.make_async_copy(
                lut_hbm.at[obfw_v.at[ph, pl.ds(bi * _L + l0, l1 - l0)]],
                char_v.at[ph, bi, pl.ds(l0, l1 - l0)], gsem)
            for bi in range(_NB)
            for l0, l1 in zip(_GSPLIT[:-1], _GSPLIT[1:])
        ]

    for d in in_copies(0, 0):
        d.start()

    def do_block(blk, ph):
        @pl.when(blk >= 2)
        def _():
            for d in out_copies(blk - 2, ph):
                d.wait()
        for d in in_copies(blk, ph):
            d.wait()
        for d in in_copies(jnp.minimum(blk + 1, _NBLK - 1), 1 - ph):
            d.start()

        base = wid * _CHUNK + blk * _K

        @plsc.parallel_loop(0, _NGRP, step=1, unroll=4)
        def grp_body(i):
            o = i * 16
            p = pos_v[ph, pl.ds(o, 16)]
            k_a0 = plsc.bitcast(plsc.load_gather(a0_v, [p]), jnp.uint32)
            k_a1 = plsc.bitcast(plsc.load_gather(a1_v, [p]), jnp.uint32)
            k_b0 = plsc.bitcast(plsc.load_gather(b0_v, [p]), jnp.uint32)
            k_b1 = plsc.bitcast(plsc.load_gather(b1_v, [p]), jnp.uint32)
            prg = plsc.load_gather(pri_v, [p])
            cnt = plsc.bitcast(
                base + o + lax.broadcasted_iota(jnp.int32, (16,), 0),
                jnp.uint32)
            hi = _tf_xor(k_a0, k_a1, cnt)
            lo = _tf_xor(k_b0, k_b1, cnt)
            off = lax.rem(lax.rem(hi, _SPAN) * _MULT + lax.rem(lo, _SPAN),
                          _SPAN)
            idxf = p * jnp.int32(_M) + plsc.bitcast(off, jnp.int32)
            w = plsc.load_gather(tgt_v, [idxf])
            obfw_v[ph, pl.ds(o, 16)] = w
            prim_v[ph, pl.ds(o, 16)] = prg

        @pl.when(blk >= 1)
        def _():
            for d in gather_copies(1 - ph):
                d.wait()
            for d in out_copies(blk - 1, 1 - ph):
                d.start()
        for d in gather_copies(ph):
            d.start()

    def body2(g, carry):
        do_block(g * 2, 0)
        do_block(g * 2 + 1, 1)
        return carry

    lax.fori_loop(0, _NBLK // 2, body2, 0)
    for d in gather_copies(1):
        d.wait()
    for d in out_copies(_NBLK - 1, 1):
        d.start()
    for ph in (0, 1):
        for d in out_copies(_NBLK - 2 + ph, ph):
            d.wait()
    for d in in_copies(_NBLK - 1, 0):
        d.wait()


@functools.cache
def _sc_call():
  return pl.kernel(
    _sc_body,
    mesh=plsc.VectorSubcoreMesh(core_axis_name="c", subcore_axis_name="s",
                                num_cores=_NC, num_subcores=_NS),
    out_type=[
        jax.ShapeDtypeStruct((_N,), jnp.int32),
        jax.ShapeDtypeStruct((_B, _L, 128), jnp.int32),
        jax.ShapeDtypeStruct((_N,), jnp.int32),
    ],
    scratch_types=[
        pltpu.VMEM((_TAB,), jnp.int32),
        pltpu.VMEM((_TAB,), jnp.int32),
        pltpu.VMEM((_TAB,), jnp.int32),
        pltpu.VMEM((_TAB,), jnp.int32),
        pltpu.VMEM((_TAB,), jnp.int32),
        pltpu.VMEM((_P * _M,), jnp.int32),
        pltpu.VMEM((2, _K), jnp.int32),
        pltpu.VMEM((2, _K), jnp.int32),
        pltpu.VMEM((2, _K), jnp.int32),
        pltpu.VMEM((2, _NB, _L, _C), jnp.int32),
        pltpu.SemaphoreType.DMA,
        pltpu.SemaphoreType.DMA,
        pltpu.SemaphoreType.DMA,
        pltpu.SemaphoreType.DMA,
    ],
    compiler_params=pltpu.CompilerParams(needs_layout_passes=False,
                                         use_tc_tiling_on_sc=False),
  )


def _sc_run(pos_f, a0, a1, b0, b1, pri_tab, tgt_f, lut):
    return _sc_call()(pos_f, a0, a1, b0, b1, pri_tab, tgt_f, lut)


def kernel(inp_word, inp_char, inp_pos, inp_mask, tgtwords, pri_term_mask, lut):
    pos_f = inp_pos.reshape(_N)
    pri_tab = jnp.zeros((_TAB,), jnp.int32).at[:_P].set(
        pri_term_mask.astype(jnp.int32))
    tgt_f = tgtwords.reshape(_P * _M)

    obfw_f, char_f, prim_f = _sc_run(
        pos_f, jnp.asarray(_A0), jnp.asarray(_A1), jnp.asarray(_B0),
        jnp.asarray(_B1), pri_tab, tgt_f, lut)

    obf_word = obfw_f.reshape(_B, _L)
    obf_char = char_f[:, :, :_C]
    pri_mask = prim_f.reshape(_B, _L).astype(bool)
    cpy_mask = inp_mask & (inp_word == obf_word)
    obf_mask = jnp.ones((_B, _L), bool)
    return (obf_word, inp_word, obf_char, inp_pos, obf_mask, pri_mask,
            cpy_mask)


# speedup vs baseline: 1.8371x; 1.0104x over previous
import functools

import numpy as np
import jax
import jax.numpy as jnp
from jax import lax
from jax.experimental import pallas as pl
from jax.experimental.pallas import tpu as pltpu
from jax.experimental.pallas import tpu_sc as plsc

_B, _L, _C = 4096, 200, 16
_V, _P, _M = 100000, 45, 2000
_N = _B * _L
_NC, _NS = 2, 16
_NW = _NC * _NS
_CHUNK = _N // _NW
_BROWS = _B // _NW
_NB = 4
_K = _NB * _L
_NBLK = _BROWS // _NB
_GSPLIT = (0, 128, 200)
_NGRP = _K // 16

_SPAN = np.uint32(_M)
_MULT = np.uint32((pow(2, 16, _M) ** 2) % _M)
_TAB = 64


def _np_threefry_block(k0, k1, x0, x1):
    k0 = np.uint32(k0)
    k1 = np.uint32(k1)
    ks = [k0, k1, np.uint32(k0 ^ k1 ^ np.uint32(0x1BD11BDA))]
    rots = [(13, 15, 26, 6), (17, 29, 16, 24)]
    x = [np.uint32(x0 + ks[0]), np.uint32(x1 + ks[1])]
    for g in range(5):
        for r in rots[g % 2]:
            x[0] = np.uint32(x[0] + x[1])
            x[1] = np.uint32((np.uint32(x[1] << np.uint32(r))) |
                             (x[1] >> np.uint32(32 - r)))
            x[1] = np.uint32(x[0] ^ x[1])
        x[0] = np.uint32(x[0] + ks[(g + 1) % 3])
        x[1] = np.uint32(x[1] + ks[(g + 2) % 3] + np.uint32(g + 1))
    return x[0], x[1]


def _make_key_tables():
    a0 = np.zeros(_TAB, np.uint32)
    a1 = np.zeros(_TAB, np.uint32)
    b0 = np.zeros(_TAB, np.uint32)
    b1 = np.zeros(_TAB, np.uint32)
    for p in range(_P):
        f0, f1 = _np_threefry_block(0, 42, 0, p)
        a0[p], a1[p] = _np_threefry_block(f0, f1, 0, 0)
        b0[p], b1[p] = _np_threefry_block(f0, f1, 0, 1)
    return (a0.view(np.int32), a1.view(np.int32),
            b0.view(np.int32), b1.view(np.int32))


_A0, _A1, _B0, _B1 = _make_key_tables()


def _tf_xor(k0, k1, cnt):
    ks = (k0, k1, k0 ^ k1 ^ jnp.uint32(0x1BD11BDA))
    x0 = k0
    x1 = cnt + k1
    rots = ((13, 15, 26, 6), (17, 29, 16, 24))
    for g in range(5):
        for r in rots[g % 2]:
            x0 = x0 + x1
            x1 = lax.shift_left(x1, jnp.uint32(r)) | lax.shift_right_logical(
                x1, jnp.uint32(32 - r))
            x1 = x0 ^ x1
        x0 = x0 + ks[(g + 1) % 3]
        x1 = x1 + ks[(g + 2) % 3] + jnp.uint32(g + 1)
    return x0 ^ x1


def _sc_body(pos_hbm, a0_hbm, a1_hbm, b0_hbm, b1_hbm, pri_hbm,
             tgt_hbm, lut_hbm,
             obfw_hbm, char_hbm, prim_hbm,
             a0_v, a1_v, b0_v, b1_v, pri_v, tgt_v,
             pos_v, obfw_v, prim_v, char_v,
             isem, gsem, osem0, osem1):
    wid = lax.axis_index("s") * _NC + lax.axis_index("c")
    osems = (osem0, osem1)

    pltpu.sync_copy(a0_hbm, a0_v)
    pltpu.sync_copy(a1_hbm, a1_v)
    pltpu.sync_copy(b0_hbm, b0_v)
    pltpu.sync_copy(b1_hbm, b1_v)
    pltpu.sync_copy(pri_hbm, pri_v)
    pltpu.sync_copy(tgt_hbm, tgt_v)

    def in_copies(blk, ph):
        base = wid * _CHUNK + blk * _K
        return (
            pltpu.make_async_copy(pos_hbm.at[pl.ds(base, _K)],
                                  pos_v.at[ph], isem),
        )

    def out_copies(blk, ph):
        base = wid * _CHUNK + blk * _K
        b0 = wid * _BROWS + blk * _NB
        return (
            pltpu.make_async_copy(obfw_v.at[ph],
                                  obfw_hbm.at[pl.ds(base, _K)], osems[ph]),
            pltpu.make_async_copy(
                char_v.at[ph],
                char_hbm.at[pl.ds(b0, _NB), :, pl.ds(0, _C)], osems[ph]),
            pltpu.make_async_copy(prim_v.at[ph],
                                  prim_hbm.at[pl.ds(base, _K)], osems[ph]),
        )

    def gather_copies(ph):
        return [
            pltpu.make_async_copy(
                lut_hbm.at[obfw_v.at[ph, pl.ds(bi * _L + l0, l1 - l0)]],
                char_v.at[ph, bi, pl.ds(l0, l1 - l0)], gsem)
            for bi in range(_NB)
            for l0, l1 in zip(_GSPLIT[:-1], _GSPLIT[1:])
        ]

    for d in in_copies(0, 0):
        d.start()

    def do_block(blk, ph):
        @pl.when(blk >= 2)
        def _():
            for d in out_copies(blk - 2, ph):
                d.wait()
        for d in in_copies(blk, ph):
            d.wait()
        for d in in_copies(jnp.minimum(blk + 1, _NBLK - 1), 1 - ph):
            d.start()

        base = wid * _CHUNK + blk * _K

        @plsc.parallel_loop(0, _NGRP, step=1, unroll=5)
        def grp_body(i):
            o = i * 16
            p = pos_v[ph, pl.ds(o, 16)]
            k_a0 = plsc.bitcast(plsc.load_gather(a0_v, [p]), jnp.uint32)
            k_a1 = plsc.bitcast(plsc.load_gather(a1_v, [p]), jnp.uint32)
            k_b0 = plsc.bitcast(plsc.load_gather(b0_v, [p]), jnp.uint32)
            k_b1 = plsc.bitcast(plsc.load_gather(b1_v, [p]), jnp.uint32)
            prg = plsc.load_gather(pri_v, [p])
            cnt = plsc.bitcast(
                base + o + lax.broadcasted_iota(jnp.int32, (16,), 0),
                jnp.uint32)
            hi = _tf_xor(k_a0, k_a1, cnt)
            lo = _tf_xor(k_b0, k_b1, cnt)
            off = lax.rem(lax.rem(hi, _SPAN) * _MULT + lax.rem(lo, _SPAN),
                          _SPAN)
            idxf = p * jnp.int32(_M) + plsc.bitcast(off, jnp.int32)
            w = plsc.load_gather(tgt_v, [idxf])
            obfw_v[ph, pl.ds(o, 16)] = w
            prim_v[ph, pl.ds(o, 16)] = prg

        @pl.when(blk >= 1)
        def _():
            for d in gather_copies(1 - ph):
                d.wait()
            for d in out_copies(blk - 1, 1 - ph):
                d.start()
        for d in gather_copies(ph):
            d.start()

    def body2(g, carry):
        do_block(g * 2, 0)
        do_block(g * 2 + 1, 1)
        return carry

    lax.fori_loop(0, _NBLK // 2, body2, 0)
    for d in gather_copies(1):
        d.wait()
    for d in out_copies(_NBLK - 1, 1):
        d.start()
    for ph in (0, 1):
        for d in out_copies(_NBLK - 2 + ph, ph):
            d.wait()
    for d in in_copies(_NBLK - 1, 0):
        d.wait()


@functools.cache
def _sc_call():
  return pl.kernel(
    _sc_body,
    mesh=plsc.VectorSubcoreMesh(core_axis_name="c", subcore_axis_name="s",
                                num_cores=_NC, num_subcores=_NS),
    out_type=[
        jax.ShapeDtypeStruct((_N,), jnp.int32),
        jax.ShapeDtypeStruct((_B, _L, 128), jnp.int32),
        jax.ShapeDtypeStruct((_N,), jnp.int32),
    ],
    scratch_types=[
        pltpu.VMEM((_TAB,), jnp.int32),
        pltpu.VMEM((_TAB,), jnp.int32),
        pltpu.VMEM((_TAB,), jnp.int32),
        pltpu.VMEM((_TAB,), jnp.int32),
        pltpu.VMEM((_TAB,), jnp.int32),
        pltpu.VMEM((_P * _M,), jnp.int32),
        pltpu.VMEM((2, _K), jnp.int32),
        pltpu.VMEM((2, _K), jnp.int32),
        pltpu.VMEM((2, _K), jnp.int32),
        pltpu.VMEM((2, _NB, _L, _C), jnp.int32),
        pltpu.SemaphoreType.DMA,
        pltpu.SemaphoreType.DMA,
        pltpu.SemaphoreType.DMA,
        pltpu.SemaphoreType.DMA,
    ],
    compiler_params=pltpu.CompilerParams(needs_layout_passes=False,
                                         use_tc_tiling_on_sc=False),
  )


def _sc_run(pos_f, a0, a1, b0, b1, pri_tab, tgt_f, lut):
    return _sc_call()(pos_f, a0, a1, b0, b1, pri_tab, tgt_f, lut)


def kernel(inp_word, inp_char, inp_pos, inp_mask, tgtwords, pri_term_mask, lut):
    pos_f = inp_pos.reshape(_N)
    pri_tab = jnp.zeros((_TAB,), jnp.int32).at[:_P].set(
        pri_term_mask.astype(jnp.int32))
    tgt_f = tgtwords.reshape(_P * _M)

    obfw_f, char_f, prim_f = _sc_run(
        pos_f, jnp.asarray(_A0), jnp.asarray(_A1), jnp.asarray(_B0),
        jnp.asarray(_B1), pri_tab, tgt_f, lut)

    obf_word = obfw_f.reshape(_B, _L)
    obf_char = char_f[:, :, :_C]
    pri_mask = prim_f.reshape(_B, _L).astype(bool)
    cpy_mask = inp_mask & (inp_word == obf_word)
    obf_mask = jnp.ones((_B, _L), bool)
    return (obf_word, inp_word, obf_char, inp_pos, obf_mask, pri_mask,
            cpy_mask)
